# baseline (speedup 1.0000x reference)
import functools

import jax
import jax.numpy as jnp
from jax import lax
from jax.experimental import pallas as pl
from jax.experimental.pallas import tpu as pltpu
from jax.experimental.pallas import tpu_sc as plsc

D = 64
BLK = 128


def _build(B, K, NC, NS):
    NW = NC * NS
    units = K * (B // BLK)
    U = units // NW
    assert U * NW == units and U % 2 == 0
    n_c = B // BLK

    mesh = plsc.VectorSubcoreMesh(core_axis_name="c", subcore_axis_name="s")

    @functools.partial(
        pl.kernel,
        out_type=jax.ShapeDtypeStruct((K, D // 8, n_c, 8, BLK), jnp.float32),
        mesh=mesh,
        compiler_params=pltpu.CompilerParams(use_tc_tiling_on_sc=False,
                                             needs_layout_passes=False),
        scratch_types=[
            pltpu.VMEM((U, 2 * BLK), jnp.int32),
            pltpu.VMEM((2 * BLK, 32), jnp.float32),
            pltpu.VMEM((2 * BLK, 32), jnp.float32),
            pltpu.VMEM((8, 8, BLK), jnp.float32),
            pltpu.VMEM((8, 8, BLK), jnp.float32),
            pltpu.SemaphoreType.DMA,
            pltpu.SemaphoreType.DMA,
            pltpu.SemaphoreType.DMA,
            pltpu.SemaphoreType.DMA,
        ],
    )
    def gather_kernel(table_hbm, idx_hbm, out_hbm, idx_v, g0, g1, t0, t1,
                      gs0, gs1, os0, os1):
        gbuf = (g0, g1)
        tbuf = (t0, t1)
        gsem = (gs0, gs1)
        osem = (os0, os1)

        wid = lax.axis_index("s") * NC + lax.axis_index("c")
        u0 = wid * U

        pltpu.sync_copy(idx_hbm.at[pl.ds(u0, U)], idx_v)

        def fire_gather(b, u):
            pltpu.make_async_copy(
                table_hbm.at[idx_v.at[u]], gbuf[b], gsem[b]).start()

        def wait_gather(b):
            pltpu.make_async_copy(
                table_hbm.at[idx_v.at[0]], gbuf[b], gsem[b]).wait()

        def fire_flush(b, u):
            g = u0 + u
            k = g // n_c
            c = g % n_c
            pltpu.make_async_copy(
                tbuf[b], out_hbm.at[k, pl.ds(0, 8), c], osem[b]).start()

        def wait_flush(b):
            pltpu.make_async_copy(
                tbuf[b], out_hbm.at[0, pl.ds(0, 8), 0], osem[b]).wait()

        iota2 = lax.iota(jnp.int32, 16) * 2

        def transpose(b):
            g = gbuf[b]
            t = tbuf[b]
            for q in range(8):
                r0 = iota2 + (32 * q)
                r1 = r0 + 1
                for m in range(32):
                    cm = jnp.full((16,), m, jnp.int32)
                    t[m >> 3, m & 7, pl.ds(16 * q, 16)] = (
                        plsc.load_gather(g, [r0, cm]))
                    t[(m + 32) >> 3, m & 7, pl.ds(16 * q, 16)] = (
                        plsc.load_gather(g, [r1, cm]))

        fire_gather(0, 0)

        def body(i, carry):
            for b in (0, 1):
                u = 2 * i + b

                @pl.when(u + 1 < U)
                def _():
                    fire_gather(1 - b, u + 1)

                wait_gather(b)

                @pl.when(u >= 2)
                def _():
                    wait_flush(b)

                transpose(b)
                fire_flush(b, u)
            return carry

        lax.fori_loop(0, U // 2, body, 0)
        wait_flush(0)
        wait_flush(1)

    return gather_kernel


def kernel(input, indices):
    B, K = indices.shape
    info = plsc.get_sparse_core_info()
    NC, NS = info.num_cores, info.num_subcores

    table2 = input.reshape(2 * input.shape[0], 32)
    idxT = indices.T.astype(jnp.int32)
    d2 = (idxT.reshape(K, B // BLK, BLK, 1) * 2
          + jnp.arange(2, dtype=jnp.int32)).reshape(K * (B // BLK), 2 * BLK)

    out5 = _build(B, K, NC, NS)(table2, d2)
    return out5.transpose(2, 4, 0, 1, 3).reshape(B, K, D)


# speedup vs baseline: 2.1515x; 2.1515x over previous
import functools

import jax
import jax.numpy as jnp
from jax import lax
from jax.experimental import pallas as pl
from jax.experimental.pallas import tpu as pltpu
from jax.experimental.pallas import tpu_sc as plsc

D = 64
BPU = 2
KPAD, DPAD = 56, 128


def _build(B, K, NC, NS):
    NW = NC * NS
    ROWS = BPU * K
    assert ROWS <= 128
    U = B // BPU // NW
    assert U * BPU * NW == B and U % 2 == 0

    mesh = plsc.VectorSubcoreMesh(core_axis_name="c", subcore_axis_name="s")

    @functools.partial(
        pl.kernel,
        out_type=jax.ShapeDtypeStruct((B, KPAD, DPAD), jnp.float32),
        mesh=mesh,
        compiler_params=pltpu.CompilerParams(use_tc_tiling_on_sc=False),
        scratch_types=[
            pltpu.VMEM((U, ROWS), jnp.int32),
            pltpu.VMEM((ROWS, D), jnp.float32),
            pltpu.VMEM((ROWS, D), jnp.float32),
            pltpu.SemaphoreType.DMA,
            pltpu.SemaphoreType.DMA,
            pltpu.SemaphoreType.DMA,
            pltpu.SemaphoreType.DMA,
        ],
    )
    def gather_kernel(table_hbm, idx_hbm, out_hbm, idx_v, g0, g1,
                      gs0, gs1, os0, os1):
        gbuf = (g0, g1)
        gsem = (gs0, gs1)
        osem = (os0, os1)

        wid = lax.axis_index("s") * NC + lax.axis_index("c")
        b0 = wid * U * BPU

        pltpu.sync_copy(idx_hbm.at[pl.ds(wid * U, U)], idx_v)

        def fire_gather(b, u):
            pltpu.make_async_copy(
                table_hbm.at[idx_v.at[u]], gbuf[b], gsem[b]).start()

        def wait_gather(b):
            pltpu.make_async_copy(
                table_hbm.at[idx_v.at[0]], gbuf[b], gsem[b]).wait()

        def fire_flush(b, u):
            for j in range(BPU):
                pltpu.make_async_copy(
                    gbuf[b].at[pl.ds(j * K, K)],
                    out_hbm.at[b0 + u * BPU + j, pl.ds(0, K), pl.ds(0, D)],
                    osem[b]).start()

        def wait_flush(b):
            for j in range(BPU):
                pltpu.make_async_copy(
                    gbuf[b].at[pl.ds(0, K)],
                    out_hbm.at[0, pl.ds(0, K), pl.ds(0, D)],
                    osem[b]).wait()

        fire_gather(0, 0)

        def body(i, carry):
            for b in (0, 1):
                u = 2 * i + b

                @pl.when(u + 1 < U)
                def _():
                    fire_gather(1 - b, u + 1)

                wait_gather(b)

                @pl.when(u >= 2)
                def _():
                    wait_flush(b)

                fire_flush(b, u)
            return carry

        lax.fori_loop(0, U // 2, body, 0)
        wait_flush(0)
        wait_flush(1)

    return gather_kernel


def kernel(input, indices):
    B, K = indices.shape
    info = plsc.get_sparse_core_info()
    NC, NS = info.num_cores, info.num_subcores

    idx2d = indices.astype(jnp.int32).reshape(B // BPU, BPU * K)
    out = _build(B, K, NC, NS)(input, idx2d)
    return out[:, :K, :D]
